# baseline (speedup 1.0000x reference)
import jax
import jax.numpy as jnp
from jax.experimental import pallas as pl
from jax.experimental.pallas import tpu as pltpu


def _fused_kernel(feat_ref, wt_ref, adj_ref, out_ref, fr_ref):
    @pl.when(pl.program_id(0) == 0)
    def _():
        fr_ref[:] = jnp.dot(feat_ref[:], wt_ref[:], preferred_element_type=jnp.float32)

    acc = jnp.dot(adj_ref[:], fr_ref[:], preferred_element_type=jnp.float32)
    out_ref[:] = jnp.maximum(acc, 0.0)


def kernel(feat, adj, W):
    n, in_ft = feat.shape
    out_ft = W.shape[0]

    bm = 200
    out = pl.pallas_call(
        _fused_kernel,
        grid=(n // bm,),
        in_specs=[
            pl.BlockSpec((n, in_ft), lambda i: (0, 0)),
            pl.BlockSpec((in_ft, out_ft), lambda i: (0, 0)),
            pl.BlockSpec((bm, n), lambda i: (i, 0)),
        ],
        out_specs=pl.BlockSpec((bm, out_ft), lambda i: (i, 0)),
        out_shape=jax.ShapeDtypeStruct((n, out_ft), jnp.float32),
        scratch_shapes=[pltpu.VMEM((n, out_ft), jnp.float32)],
        compiler_params=pltpu.CompilerParams(
            dimension_semantics=("arbitrary",),
        ),
    )(feat, W.T, adj)
    return out


# speedup vs baseline: 1.0033x; 1.0033x over previous
import jax
import jax.numpy as jnp
from jax.experimental import pallas as pl
from jax.experimental.pallas import tpu as pltpu


def _fused_kernel(feat_ref, wt_ref, adj_ref, out_ref, fr_ref):
    @pl.when(pl.program_id(0) == 0)
    def _():
        fr_ref[:] = jnp.dot(feat_ref[:], wt_ref[:], preferred_element_type=jnp.float32)

    acc = jnp.dot(adj_ref[:], fr_ref[:], preferred_element_type=jnp.float32)
    out_ref[:] = jnp.maximum(acc, 0.0)


def kernel(feat, adj, W):
    n, in_ft = feat.shape
    out_ft = W.shape[0]

    bm = 400
    out = pl.pallas_call(
        _fused_kernel,
        grid=(n // bm,),
        in_specs=[
            pl.BlockSpec((n, in_ft), lambda i: (0, 0)),
            pl.BlockSpec((in_ft, out_ft), lambda i: (0, 0)),
            pl.BlockSpec((bm, n), lambda i: (i, 0)),
        ],
        out_specs=pl.BlockSpec((bm, out_ft), lambda i: (i, 0)),
        out_shape=jax.ShapeDtypeStruct((n, out_ft), jnp.float32),
        scratch_shapes=[pltpu.VMEM((n, out_ft), jnp.float32)],
        compiler_params=pltpu.CompilerParams(
            dimension_semantics=("arbitrary",),
        ),
    )(feat, W.T, adj)
    return out
